# baseline (speedup 1.0000x reference)
import dataclasses
import functools

import jax
import jax.numpy as jnp
from jax import lax
from jax.experimental import pallas as pl
from jax.experimental.pallas import tpu as pltpu
from jax.experimental.pallas import tpu_sc as plsc

N = 10000
E = 320000
F_IN = 128
HID = 128
NCLS = 64

NCORE = 2
NSUB = 16
NTILE = NCORE * NSUB

CHUNK = 128
CPT = 80
SLAB = 16
EPAD = NTILE * CPT * CHUNK
CPT0 = 80
CPT1 = 80
TRASH = N
NACC = 10240
RPT = NACC // NSUB

_mesh = plsc.VectorSubcoreMesh(core_axis_name="c", subcore_axis_name="s")

_sc_params = pltpu.CompilerParams()
if "needs_layout_passes" in pltpu.CompilerParams.__dataclass_fields__:
    _sc_params = dataclasses.replace(_sc_params, needs_layout_passes=False)


def _sc_degrees(src_p, dst_p):
    @functools.partial(
        pl.kernel,
        out_type=jax.ShapeDtypeStruct((NCORE, 2, NACC), jnp.float32),
        mesh=_mesh,
        compiler_params=_sc_params,
        scratch_types=[
            pltpu.VMEM((CPT * CHUNK,), jnp.int32),
            pltpu.VMEM((CPT * CHUNK,), jnp.int32),
            pltpu.VMEM((NACC,), jnp.float32),
            pltpu.VMEM((NACC,), jnp.float32),
            pltpu.VMEM((RPT,), jnp.float32),
            pltpu.VMEM_SHARED((NSUB, NACC), jnp.float32),
            pltpu.VMEM_SHARED((NSUB, NACC), jnp.float32),
        ],
    )
    def deg_kernel(src_hbm, dst_hbm, out_hbm,
                   sidx, didx, shist, dhist, tmp, ssh, dsh):
        c = lax.axis_index("c")
        s = lax.axis_index("s")
        t = c * NSUB + s
        zeros16 = jnp.zeros((16,), jnp.float32)
        ones16 = jnp.ones((16,), jnp.float32)

        @pl.loop(0, NACC, step=16)
        def _(i):
            shist[pl.ds(i, 16)] = zeros16
            dhist[pl.ds(i, 16)] = zeros16

        base = t * CPT * CHUNK
        pltpu.sync_copy(src_hbm.at[pl.ds(base, CPT * CHUNK)], sidx)
        pltpu.sync_copy(dst_hbm.at[pl.ds(base, CPT * CHUNK)], didx)

        @pl.loop(0, CPT * CHUNK, step=16)
        def _(j):
            plsc.addupdate_scatter(shist, [sidx[pl.ds(j, 16)]], ones16)
            plsc.addupdate_scatter(dhist, [didx[pl.ds(j, 16)]], ones16)

        pltpu.sync_copy(shist, ssh.at[s])
        pltpu.sync_copy(dhist, dsh.at[s])
        plsc.subcore_barrier()

        for sh, kind, acc in ((ssh, 0, shist), (dsh, 1, dhist)):
            @pl.loop(0, NSUB)
            def _(slot):
                pltpu.sync_copy(sh.at[slot, pl.ds(s * RPT, RPT)], tmp)

                @pl.loop(0, RPT, step=16)
                def _(i):
                    @pl.when(slot == 0)
                    def _():
                        acc[pl.ds(i, 16)] = tmp[pl.ds(i, 16)]

                    @pl.when(slot != 0)
                    def _():
                        acc[pl.ds(i, 16)] = acc[pl.ds(i, 16)] + tmp[pl.ds(i, 16)]

            pltpu.sync_copy(acc.at[pl.ds(0, RPT)],
                            out_hbm.at[c, kind, pl.ds(s * RPT, RPT)])

    return deg_kernel(src_p, dst_p)


def _sc_gather_scatter(feat, src_p, dst_p, zrows, f):
    @functools.partial(
        pl.kernel,
        out_type=jax.ShapeDtypeStruct((NCORE, NACC, f), jnp.float32),
        mesh=_mesh,
        scratch_types=[
            pltpu.VMEM((CHUNK,), jnp.int32),
            pltpu.VMEM((CHUNK,), jnp.int32),
            pltpu.VMEM((CHUNK, f), jnp.float32),
            pltpu.VMEM_SHARED((NACC, f), jnp.float32),
            pltpu.SemaphoreType.DMA,
        ],
    )
    def edge_kernel(feat_hbm, src_hbm, dst_hbm, z_hbm, out_hbm,
                    sidx, didx, rows, acc, sem):
        c = lax.axis_index("c")
        s = lax.axis_index("s")

        pltpu.sync_copy(z_hbm, acc.at[pl.ds(s * RPT, RPT)])
        plsc.subcore_barrier()

        t = c * NSUB + s

        @pl.loop(0, CPT)
        def _(ch):
            base = (t * CPT + ch) * CHUNK
            pltpu.sync_copy(src_hbm.at[pl.ds(base, CHUNK)], sidx)
            pltpu.sync_copy(dst_hbm.at[pl.ds(base, CHUNK)], didx)
            pltpu.async_copy(feat_hbm.at[sidx], rows, sem).wait()
            pltpu.sync_copy(rows, acc.at[didx], add=True)

        plsc.subcore_barrier()
        pltpu.sync_copy(acc.at[pl.ds(s * RPT, RPT)],
                        out_hbm.at[c, pl.ds(s * RPT, RPT)])

    return edge_kernel(feat, src_p, dst_p, zrows)


_R = 1024


def _tc_norms_scale(x_pad, degs):
    def body(deg_ref, x_ref, feat_ref, ns_ref, nd_ref):
        d = deg_ref[...]
        p = d[0] + d[1]
        od = jnp.maximum(p[0], 1.0)
        idg = jnp.maximum(p[1], 1.0)
        ns = lax.rsqrt(od)
        nd = lax.rsqrt(idg)
        ns_ref[...] = ns
        nd_ref[...] = nd
        feat_ref[...] = x_ref[...] * ns

    return pl.pallas_call(
        body,
        grid=(NACC // _R,),
        in_specs=[
            pl.BlockSpec((NCORE, 2, _R, 1), lambda i: (0, 0, i, 0)),
            pl.BlockSpec((_R, F_IN), lambda i: (i, 0)),
        ],
        out_specs=[
            pl.BlockSpec((_R, F_IN), lambda i: (i, 0)),
            pl.BlockSpec((_R, 1), lambda i: (i, 0)),
            pl.BlockSpec((_R, 1), lambda i: (i, 0)),
        ],
        out_shape=[
            jax.ShapeDtypeStruct((NACC, F_IN), jnp.float32),
            jax.ShapeDtypeStruct((NACC, 1), jnp.float32),
            jax.ShapeDtypeStruct((NACC, 1), jnp.float32),
        ],
    )(degs, x_pad)


def _tc_layer1(agg_p, ns, nd, W1, b1, W2):
    def body(agg_ref, ns_ref, nd_ref, w1_ref, b1_ref, w2_ref, out_ref):
        a = (agg_ref[0] + agg_ref[1]) * nd_ref[...]
        h = jnp.dot(a, w1_ref[...], preferred_element_type=jnp.float32)
        h = jnp.maximum(h + b1_ref[...], 0.0)
        h = h * ns_ref[...]
        out_ref[...] = jnp.dot(h, w2_ref[...], preferred_element_type=jnp.float32)

    return pl.pallas_call(
        body,
        grid=(NACC // _R,),
        in_specs=[
            pl.BlockSpec((NCORE, _R, HID), lambda i: (0, i, 0)),
            pl.BlockSpec((_R, 1), lambda i: (i, 0)),
            pl.BlockSpec((_R, 1), lambda i: (i, 0)),
            pl.BlockSpec((HID, HID), lambda i: (0, 0)),
            pl.BlockSpec((1, HID), lambda i: (0, 0)),
            pl.BlockSpec((HID, HID), lambda i: (0, 0)),
        ],
        out_specs=pl.BlockSpec((_R, HID), lambda i: (i, 0)),
        out_shape=jax.ShapeDtypeStruct((NACC, HID), jnp.float32),
    )(agg_p, ns, nd, W1, b1, W2)


def _tc_layer2(agg_p, nd, b2, Wfc, bfc):
    def body(agg_ref, nd_ref, b2_ref, wfc_ref, bfc_ref, out_ref):
        h = (agg_ref[0] + agg_ref[1]) * nd_ref[...] + b2_ref[...]
        out_ref[...] = (
            jnp.dot(h, wfc_ref[...], preferred_element_type=jnp.float32)
            + bfc_ref[...]
        )

    return pl.pallas_call(
        body,
        grid=(NACC // _R,),
        in_specs=[
            pl.BlockSpec((NCORE, _R, HID), lambda i: (0, i, 0)),
            pl.BlockSpec((_R, 1), lambda i: (i, 0)),
            pl.BlockSpec((1, HID), lambda i: (0, 0)),
            pl.BlockSpec((HID, NCLS), lambda i: (0, 0)),
            pl.BlockSpec((1, NCLS), lambda i: (0, 0)),
        ],
        out_specs=pl.BlockSpec((_R, NCLS), lambda i: (i, 0)),
        out_shape=jax.ShapeDtypeStruct((NACC, NCLS), jnp.float32),
    )(agg_p, nd, b2, Wfc, bfc)


def kernel(x, edge_index, W1, b1, W2, b2, Wfc, bfc):
    src = edge_index[0].astype(jnp.int32)
    dst = edge_index[1].astype(jnp.int32)
    pad = jnp.full((EPAD - E,), TRASH, jnp.int32)
    src_p = jnp.concatenate([src, pad])
    dst_p = jnp.concatenate([dst, pad])

    x_pad = jnp.pad(x, ((0, NACC - N), (0, 0)))

    z128 = jnp.zeros((RPT, F_IN), jnp.float32)

    W2p = jnp.pad(W2, ((0, 0), (0, HID - NCLS)))
    b2p = jnp.pad(b2, (0, HID - NCLS)).reshape(1, HID)
    Wfcp = jnp.pad(Wfc, ((0, HID - NCLS), (0, 0)))

    degs = _sc_degrees(src_p, dst_p).reshape(NCORE, 2, NACC, 1)
    feat1, ns, nd = _tc_norms_scale(x_pad, degs)
    agg1 = _sc_gather_scatter(feat1, src_p, dst_p, z128, F_IN)
    feat2 = _tc_layer1(agg1, ns, nd, W1, b1.reshape(1, HID), W2p)
    agg2 = _sc_gather_scatter(feat2, src_p, dst_p, z128, HID)
    out = _tc_layer2(agg2, nd, b2p, Wfcp, bfc.reshape(1, NCLS))
    return out[:N]


# speedup vs baseline: 2.1960x; 2.1960x over previous
import dataclasses
import functools

import jax
import jax.numpy as jnp
from jax import lax
from jax.experimental import pallas as pl
from jax.experimental.pallas import tpu as pltpu
from jax.experimental.pallas import tpu_sc as plsc

N = 10000
E = 320000
F_IN = 128
HID = 128
NCLS = 64

NCORE = 2
NSUB = 16
NTILE = NCORE * NSUB

CHUNK = 128
CPT = 80
SLAB = 16
EPAD = NTILE * CPT * CHUNK
CPT0 = 80
CPT1 = 80
TRASH = N
NACC = 10240
RPT = NACC // NSUB

_mesh = plsc.VectorSubcoreMesh(core_axis_name="c", subcore_axis_name="s")

_sc_params = pltpu.CompilerParams()
if "needs_layout_passes" in pltpu.CompilerParams.__dataclass_fields__:
    _sc_params = dataclasses.replace(_sc_params, needs_layout_passes=False)


def _sc_degrees(src_p, dst_p):
    @functools.partial(
        pl.kernel,
        out_type=jax.ShapeDtypeStruct((NCORE, 2, NACC), jnp.float32),
        mesh=_mesh,
        compiler_params=_sc_params,
        scratch_types=[
            pltpu.VMEM((CPT * CHUNK,), jnp.int32),
            pltpu.VMEM((CPT * CHUNK,), jnp.int32),
            pltpu.VMEM((NACC,), jnp.float32),
            pltpu.VMEM((NACC,), jnp.float32),
            pltpu.VMEM((RPT,), jnp.float32),
            pltpu.VMEM_SHARED((NSUB, NACC), jnp.float32),
            pltpu.VMEM_SHARED((NSUB, NACC), jnp.float32),
        ],
    )
    def deg_kernel(src_hbm, dst_hbm, out_hbm,
                   sidx, didx, shist, dhist, tmp, ssh, dsh):
        c = lax.axis_index("c")
        s = lax.axis_index("s")
        t = c * NSUB + s
        zeros16 = jnp.zeros((16,), jnp.float32)
        ones16 = jnp.ones((16,), jnp.float32)

        @pl.loop(0, NACC, step=16)
        def _(i):
            shist[pl.ds(i, 16)] = zeros16
            dhist[pl.ds(i, 16)] = zeros16

        base = t * CPT * CHUNK
        pltpu.sync_copy(src_hbm.at[pl.ds(base, CPT * CHUNK)], sidx)
        pltpu.sync_copy(dst_hbm.at[pl.ds(base, CPT * CHUNK)], didx)

        @pl.loop(0, CPT * CHUNK, step=16)
        def _(j):
            plsc.addupdate_scatter(shist, [sidx[pl.ds(j, 16)]], ones16)
            plsc.addupdate_scatter(dhist, [didx[pl.ds(j, 16)]], ones16)

        pltpu.sync_copy(shist, ssh.at[s])
        pltpu.sync_copy(dhist, dsh.at[s])
        plsc.subcore_barrier()

        for sh, kind, acc in ((ssh, 0, shist), (dsh, 1, dhist)):
            @pl.loop(0, NSUB)
            def _(slot):
                pltpu.sync_copy(sh.at[slot, pl.ds(s * RPT, RPT)], tmp)

                @pl.loop(0, RPT, step=16)
                def _(i):
                    @pl.when(slot == 0)
                    def _():
                        acc[pl.ds(i, 16)] = tmp[pl.ds(i, 16)]

                    @pl.when(slot != 0)
                    def _():
                        acc[pl.ds(i, 16)] = acc[pl.ds(i, 16)] + tmp[pl.ds(i, 16)]

            pltpu.sync_copy(acc.at[pl.ds(0, RPT)],
                            out_hbm.at[c, kind, pl.ds(s * RPT, RPT)])

    return deg_kernel(src_p, dst_p)


def _sc_gather_scatter(feat, src_p, dst_p, zrows, f):
    @functools.partial(
        pl.kernel,
        out_type=jax.ShapeDtypeStruct((NCORE, NACC, f), jnp.float32),
        mesh=_mesh,
        scratch_types=[
            pltpu.VMEM((CHUNK,), jnp.int32),
            pltpu.VMEM((CHUNK,), jnp.int32),
            pltpu.VMEM((CHUNK, f), jnp.float32),
            pltpu.VMEM_SHARED((NACC, f), jnp.float32),
            pltpu.SemaphoreType.DMA,
        ],
    )
    def edge_kernel(feat_hbm, src_hbm, dst_hbm, z_hbm, out_hbm,
                    sidx, didx, rows, acc, sem):
        c = lax.axis_index("c")
        s = lax.axis_index("s")

        pltpu.sync_copy(z_hbm, acc.at[pl.ds(s * RPT, RPT)])
        plsc.subcore_barrier()

        t = c * NSUB + s

        @pl.loop(0, CPT)
        def _(ch):
            base = (t * CPT + ch) * CHUNK
            pltpu.sync_copy(src_hbm.at[pl.ds(base, CHUNK)], sidx)
            pltpu.sync_copy(dst_hbm.at[pl.ds(base, CHUNK)], didx)
            pltpu.async_copy(feat_hbm.at[sidx], rows, sem).wait()
            pltpu.sync_copy(rows, acc.at[didx], add=True)

        plsc.subcore_barrier()
        pltpu.sync_copy(acc.at[pl.ds(s * RPT, RPT)],
                        out_hbm.at[c, pl.ds(s * RPT, RPT)])

    return edge_kernel(feat, src_p, dst_p, zrows)


_R = 1024


def _tc_norms_scale(x_pad, degs):
    def body(deg_ref, x_ref, feat_ref, ns_ref, nd_ref):
        d = deg_ref[...]
        p = d[0] + d[1]
        od = jnp.maximum(p[0], 1.0)
        idg = jnp.maximum(p[1], 1.0)
        ns = lax.rsqrt(od)
        nd = lax.rsqrt(idg)
        ns_ref[...] = ns
        nd_ref[...] = nd
        feat_ref[...] = x_ref[...] * ns

    return pl.pallas_call(
        body,
        grid=(NACC // _R,),
        in_specs=[
            pl.BlockSpec((NCORE, 2, _R, 1), lambda i: (0, 0, i, 0)),
            pl.BlockSpec((_R, F_IN), lambda i: (i, 0)),
        ],
        out_specs=[
            pl.BlockSpec((_R, F_IN), lambda i: (i, 0)),
            pl.BlockSpec((_R, 1), lambda i: (i, 0)),
            pl.BlockSpec((_R, 1), lambda i: (i, 0)),
        ],
        out_shape=[
            jax.ShapeDtypeStruct((NACC, F_IN), jnp.float32),
            jax.ShapeDtypeStruct((NACC, 1), jnp.float32),
            jax.ShapeDtypeStruct((NACC, 1), jnp.float32),
        ],
    )(degs, x_pad)


def _tc_layer1(agg_p, ns, nd, W1, b1, W2):
    def body(agg_ref, ns_ref, nd_ref, w1_ref, b1_ref, w2_ref, out_ref):
        a = (agg_ref[0] + agg_ref[1]) * nd_ref[...]
        h = jnp.dot(a, w1_ref[...], preferred_element_type=jnp.float32)
        h = jnp.maximum(h + b1_ref[...], 0.0)
        h = h * ns_ref[...]
        out_ref[...] = jnp.dot(h, w2_ref[...], preferred_element_type=jnp.float32)

    return pl.pallas_call(
        body,
        grid=(NACC // _R,),
        in_specs=[
            pl.BlockSpec((NCORE, _R, HID), lambda i: (0, i, 0)),
            pl.BlockSpec((_R, 1), lambda i: (i, 0)),
            pl.BlockSpec((_R, 1), lambda i: (i, 0)),
            pl.BlockSpec((HID, HID), lambda i: (0, 0)),
            pl.BlockSpec((1, HID), lambda i: (0, 0)),
            pl.BlockSpec((HID, HID), lambda i: (0, 0)),
        ],
        out_specs=pl.BlockSpec((_R, HID), lambda i: (i, 0)),
        out_shape=jax.ShapeDtypeStruct((NACC, HID), jnp.float32),
    )(agg_p, ns, nd, W1, b1, W2)


def _tc_layer2(agg_p, nd, b2, Wfc, bfc):
    def body(agg_ref, nd_ref, b2_ref, wfc_ref, bfc_ref, out_ref):
        h = (agg_ref[0] + agg_ref[1]) * nd_ref[...] + b2_ref[...]
        out_ref[...] = (
            jnp.dot(h, wfc_ref[...], preferred_element_type=jnp.float32)
            + bfc_ref[...]
        )

    return pl.pallas_call(
        body,
        grid=(NACC // _R,),
        in_specs=[
            pl.BlockSpec((NCORE, _R, HID), lambda i: (0, i, 0)),
            pl.BlockSpec((_R, 1), lambda i: (i, 0)),
            pl.BlockSpec((1, HID), lambda i: (0, 0)),
            pl.BlockSpec((HID, NCLS), lambda i: (0, 0)),
            pl.BlockSpec((1, NCLS), lambda i: (0, 0)),
        ],
        out_specs=pl.BlockSpec((_R, NCLS), lambda i: (i, 0)),
        out_shape=jax.ShapeDtypeStruct((NACC, NCLS), jnp.float32),
    )(agg_p, nd, b2, Wfc, bfc)


def kernel(x, edge_index, W1, b1, W2, b2, Wfc, bfc):
    src = edge_index[0].astype(jnp.int32)
    dst = edge_index[1].astype(jnp.int32)
    pad = TRASH + (jnp.arange(EPAD - E, dtype=jnp.int32) % (NACC - N))
    src_p = jnp.concatenate([src, pad])
    dst_p = jnp.concatenate([dst, pad])

    x_pad = jnp.pad(x, ((0, NACC - N), (0, 0)))

    z128 = jnp.zeros((RPT, F_IN), jnp.float32)

    W2p = jnp.pad(W2, ((0, 0), (0, HID - NCLS)))
    b2p = jnp.pad(b2, (0, HID - NCLS)).reshape(1, HID)
    Wfcp = jnp.pad(Wfc, ((0, HID - NCLS), (0, 0)))

    degs = _sc_degrees(src_p, dst_p).reshape(NCORE, 2, NACC, 1)
    feat1, ns, nd = _tc_norms_scale(x_pad, degs)
    agg1 = _sc_gather_scatter(feat1, src_p, dst_p, z128, F_IN)
    feat2 = _tc_layer1(agg1, ns, nd, W1, b1.reshape(1, HID), W2p)
    agg2 = _sc_gather_scatter(feat2, src_p, dst_p, z128, HID)
    out = _tc_layer2(agg2, nd, b2p, Wfcp, bfc.reshape(1, NCLS))
    return out[:N]


# speedup vs baseline: 3.4106x; 1.5531x over previous
import dataclasses
import functools

import jax
import jax.numpy as jnp
from jax import lax
from jax.experimental import pallas as pl
from jax.experimental.pallas import tpu as pltpu
from jax.experimental.pallas import tpu_sc as plsc

N = 10000
E = 320000
F_IN = 128
HID = 128
NCLS = 64

NCORE = 2
NSUB = 16
NTILE = NCORE * NSUB

CHUNK = 128
CPT = 80
SLAB = 16
EPAD = NTILE * CPT * CHUNK
CPT0 = 80
CPT1 = 80
TRASH = N
NACC = 10240
RPT = NACC // NSUB

_mesh = plsc.VectorSubcoreMesh(core_axis_name="c", subcore_axis_name="s")

_sc_params = pltpu.CompilerParams()
if "needs_layout_passes" in pltpu.CompilerParams.__dataclass_fields__:
    _sc_params = dataclasses.replace(_sc_params, needs_layout_passes=False)


def _sc_degrees(src_p, dst_p):
    @functools.partial(
        pl.kernel,
        out_type=jax.ShapeDtypeStruct((NCORE, 2, NACC), jnp.float32),
        mesh=_mesh,
        compiler_params=_sc_params,
        scratch_types=[
            pltpu.VMEM((CPT * CHUNK,), jnp.int32),
            pltpu.VMEM((CPT * CHUNK,), jnp.int32),
            pltpu.VMEM((NACC,), jnp.float32),
            pltpu.VMEM((NACC,), jnp.float32),
            pltpu.VMEM((RPT,), jnp.float32),
            pltpu.VMEM_SHARED((NSUB, NACC), jnp.float32),
            pltpu.VMEM_SHARED((NSUB, NACC), jnp.float32),
        ],
    )
    def deg_kernel(src_hbm, dst_hbm, out_hbm,
                   sidx, didx, shist, dhist, tmp, ssh, dsh):
        c = lax.axis_index("c")
        s = lax.axis_index("s")
        t = c * NSUB + s
        zeros16 = jnp.zeros((16,), jnp.float32)
        ones16 = jnp.ones((16,), jnp.float32)

        @pl.loop(0, NACC, step=16)
        def _(i):
            shist[pl.ds(i, 16)] = zeros16
            dhist[pl.ds(i, 16)] = zeros16

        base = t * CPT * CHUNK
        pltpu.sync_copy(src_hbm.at[pl.ds(base, CPT * CHUNK)], sidx)
        pltpu.sync_copy(dst_hbm.at[pl.ds(base, CPT * CHUNK)], didx)

        @pl.loop(0, CPT * CHUNK, step=16)
        def _(j):
            plsc.addupdate_scatter(shist, [sidx[pl.ds(j, 16)]], ones16)
            plsc.addupdate_scatter(dhist, [didx[pl.ds(j, 16)]], ones16)

        pltpu.sync_copy(shist, ssh.at[s])
        pltpu.sync_copy(dhist, dsh.at[s])
        plsc.subcore_barrier()

        for sh, kind, acc in ((ssh, 0, shist), (dsh, 1, dhist)):
            @pl.loop(0, NSUB)
            def _(slot):
                pltpu.sync_copy(sh.at[slot, pl.ds(s * RPT, RPT)], tmp)

                @pl.loop(0, RPT, step=16)
                def _(i):
                    @pl.when(slot == 0)
                    def _():
                        acc[pl.ds(i, 16)] = tmp[pl.ds(i, 16)]

                    @pl.when(slot != 0)
                    def _():
                        acc[pl.ds(i, 16)] = acc[pl.ds(i, 16)] + tmp[pl.ds(i, 16)]

            pltpu.sync_copy(acc.at[pl.ds(0, RPT)],
                            out_hbm.at[c, kind, pl.ds(s * RPT, RPT)])

    return deg_kernel(src_p, dst_p)


def _sc_gather_scatter(feat, src_p, dst_p, zrows, f):
    @functools.partial(
        pl.kernel,
        out_type=jax.ShapeDtypeStruct((NCORE, NACC, f), jnp.float32),
        mesh=_mesh,
        scratch_types=[
            pltpu.VMEM((SLAB, CHUNK), jnp.int32),
            pltpu.VMEM((SLAB, CHUNK), jnp.int32),
            pltpu.VMEM((CHUNK, f), jnp.float32),
            pltpu.VMEM((CHUNK, f), jnp.float32),
            pltpu.VMEM_SHARED((NACC, f), jnp.float32),
            pltpu.SemaphoreType.DMA,
        ],
    )
    def edge_kernel(feat_hbm, src_hbm, dst_hbm, z_hbm, out_hbm,
                    sidx, didx, rows0, rows1, acc, sem):
        c = lax.axis_index("c")
        s = lax.axis_index("s")
        t = c * NSUB + s

        pltpu.sync_copy(z_hbm, acc.at[pl.ds(s * RPT, RPT)])
        plsc.subcore_barrier()

        @pl.loop(0, CPT // SLAB)
        def _(sl):
            pltpu.sync_copy(src_hbm.at[pl.ds(t * CPT + sl * SLAB, SLAB)], sidx)
            pltpu.sync_copy(dst_hbm.at[pl.ds(t * CPT + sl * SLAB, SLAB)], didx)
            pltpu.async_copy(feat_hbm.at[sidx.at[0]], rows0, sem)

            @pl.loop(0, SLAB, step=2)
            def _(j):
                pltpu.make_async_copy(feat_hbm.at[sidx.at[j]], rows0, sem).wait()
                pltpu.async_copy(feat_hbm.at[sidx.at[j + 1]], rows1, sem)
                pltpu.sync_copy(rows0, acc.at[didx.at[j]], add=True)
                pltpu.make_async_copy(feat_hbm.at[sidx.at[j]], rows1, sem).wait()

                @pl.when(j + 2 < SLAB)
                def _():
                    pltpu.async_copy(feat_hbm.at[sidx.at[j + 2]], rows0, sem)

                pltpu.sync_copy(rows1, acc.at[didx.at[j + 1]], add=True)

        plsc.subcore_barrier()
        pltpu.sync_copy(acc.at[pl.ds(s * RPT, RPT)],
                        out_hbm.at[c, pl.ds(s * RPT, RPT)])

    return edge_kernel(feat, src_p, dst_p, zrows)


_R = 1024


def _tc_norms_scale(x_pad, degs):
    def body(deg_ref, x_ref, feat_ref, ns_ref, nd_ref):
        d = deg_ref[...]
        p = d[0] + d[1]
        od = jnp.maximum(p[0], 1.0)
        idg = jnp.maximum(p[1], 1.0)
        ns = lax.rsqrt(od)
        nd = lax.rsqrt(idg)
        ns_ref[...] = ns
        nd_ref[...] = nd
        feat_ref[...] = x_ref[...] * ns

    return pl.pallas_call(
        body,
        grid=(NACC // _R,),
        in_specs=[
            pl.BlockSpec((NCORE, 2, _R, 1), lambda i: (0, 0, i, 0)),
            pl.BlockSpec((_R, F_IN), lambda i: (i, 0)),
        ],
        out_specs=[
            pl.BlockSpec((_R, F_IN), lambda i: (i, 0)),
            pl.BlockSpec((_R, 1), lambda i: (i, 0)),
            pl.BlockSpec((_R, 1), lambda i: (i, 0)),
        ],
        out_shape=[
            jax.ShapeDtypeStruct((NACC, F_IN), jnp.float32),
            jax.ShapeDtypeStruct((NACC, 1), jnp.float32),
            jax.ShapeDtypeStruct((NACC, 1), jnp.float32),
        ],
    )(degs, x_pad)


def _tc_layer1(agg_p, ns, nd, W1, b1, W2):
    def body(agg_ref, ns_ref, nd_ref, w1_ref, b1_ref, w2_ref, out_ref):
        a = (agg_ref[0] + agg_ref[1]) * nd_ref[...]
        h = jnp.dot(a, w1_ref[...], preferred_element_type=jnp.float32)
        h = jnp.maximum(h + b1_ref[...], 0.0)
        h = h * ns_ref[...]
        out_ref[...] = jnp.dot(h, w2_ref[...], preferred_element_type=jnp.float32)

    return pl.pallas_call(
        body,
        grid=(NACC // _R,),
        in_specs=[
            pl.BlockSpec((NCORE, _R, HID), lambda i: (0, i, 0)),
            pl.BlockSpec((_R, 1), lambda i: (i, 0)),
            pl.BlockSpec((_R, 1), lambda i: (i, 0)),
            pl.BlockSpec((HID, HID), lambda i: (0, 0)),
            pl.BlockSpec((1, HID), lambda i: (0, 0)),
            pl.BlockSpec((HID, HID), lambda i: (0, 0)),
        ],
        out_specs=pl.BlockSpec((_R, HID), lambda i: (i, 0)),
        out_shape=jax.ShapeDtypeStruct((NACC, HID), jnp.float32),
    )(agg_p, ns, nd, W1, b1, W2)


def _tc_layer2(agg_p, nd, b2, Wfc, bfc):
    def body(agg_ref, nd_ref, b2_ref, wfc_ref, bfc_ref, out_ref):
        h = (agg_ref[0] + agg_ref[1]) * nd_ref[...] + b2_ref[...]
        out_ref[...] = (
            jnp.dot(h, wfc_ref[...], preferred_element_type=jnp.float32)
            + bfc_ref[...]
        )

    return pl.pallas_call(
        body,
        grid=(NACC // _R,),
        in_specs=[
            pl.BlockSpec((NCORE, _R, HID), lambda i: (0, i, 0)),
            pl.BlockSpec((_R, 1), lambda i: (i, 0)),
            pl.BlockSpec((1, HID), lambda i: (0, 0)),
            pl.BlockSpec((HID, NCLS), lambda i: (0, 0)),
            pl.BlockSpec((1, NCLS), lambda i: (0, 0)),
        ],
        out_specs=pl.BlockSpec((_R, NCLS), lambda i: (i, 0)),
        out_shape=jax.ShapeDtypeStruct((NACC, NCLS), jnp.float32),
    )(agg_p, nd, b2, Wfc, bfc)


def kernel(x, edge_index, W1, b1, W2, b2, Wfc, bfc):
    src = edge_index[0].astype(jnp.int32)
    dst = edge_index[1].astype(jnp.int32)
    pad = TRASH + (jnp.arange(EPAD - E, dtype=jnp.int32) % (NACC - N))
    src_p = jnp.concatenate([src, pad])
    dst_p = jnp.concatenate([dst, pad])

    x_pad = jnp.pad(x, ((0, NACC - N), (0, 0)))

    z128 = jnp.zeros((RPT, F_IN), jnp.float32)

    W2p = jnp.pad(W2, ((0, 0), (0, HID - NCLS)))
    b2p = jnp.pad(b2, (0, HID - NCLS)).reshape(1, HID)
    Wfcp = jnp.pad(Wfc, ((0, HID - NCLS), (0, 0)))

    src2d = src_p.reshape(NTILE * CPT, CHUNK)
    dst2d = dst_p.reshape(NTILE * CPT, CHUNK)

    degs = _sc_degrees(src_p, dst_p).reshape(NCORE, 2, NACC, 1)
    feat1, ns, nd = _tc_norms_scale(x_pad, degs)
    agg1 = _sc_gather_scatter(feat1, src2d, dst2d, z128, F_IN)
    feat2 = _tc_layer1(agg1, ns, nd, W1, b1.reshape(1, HID), W2p)
    agg2 = _sc_gather_scatter(feat2, src2d, dst2d, z128, HID)
    out = _tc_layer2(agg2, nd, b2p, Wfcp, bfc.reshape(1, NCLS))
    return out[:N]
